# Initial kernel scaffold; baseline (speedup 1.0000x reference)
import functools

import jax
import jax.numpy as jnp
from jax import lax
from jax.experimental import pallas as pl
from jax.experimental.pallas import tpu as pltpu
from jax.experimental.pallas import tpu_sc as plsc

N_NODES = 10000
IN_DIM = 128
HIDDEN = 64
NUM_CLASSES = 7
N_EDGES = 320000

NC = 2
NS = 16
NW = NC * NS
EPW = N_EDGES // NW
BATCH = 80
NB = EPW // BATCH
RPT = N_NODES // NS
DEGW = 16

_mesh = plsc.VectorSubcoreMesh(
    core_axis_name="c", subcore_axis_name="s", num_cores=NC, num_subcores=NS)


@functools.partial(
    pl.kernel,
    out_type=jax.ShapeDtypeStruct((NC, N_NODES, DEGW), jnp.float32),
    mesh=_mesh,
    scratch_types=[
        pltpu.VMEM((NB, BATCH), jnp.int32),
        pltpu.VMEM((BATCH, DEGW), jnp.float32),
        pltpu.VMEM_SHARED((N_NODES, DEGW), jnp.float32),
    ],
)
def _sc_degree(dst_hbm, ones_hbm, zeros_hbm, out_hbm, idx_v, ones_v, acc_sh):
    cid = lax.axis_index("c")
    sid = lax.axis_index("s")
    wid = sid * NC + cid
    pltpu.sync_copy(dst_hbm.at[wid], idx_v)
    pltpu.sync_copy(ones_hbm, ones_v)
    rows = pl.ds(sid * RPT, RPT)
    pltpu.sync_copy(zeros_hbm.at[rows], acc_sh.at[rows])
    plsc.subcore_barrier()

    def body(i, carry):
        pltpu.sync_copy(ones_v, acc_sh.at[idx_v.at[i]], add=True)
        return carry

    lax.fori_loop(0, NB, body, 0)
    plsc.subcore_barrier()
    pltpu.sync_copy(acc_sh.at[rows], out_hbm.at[cid, rows])


@functools.partial(
    pl.kernel,
    out_type=jax.ShapeDtypeStruct((NC, N_NODES, HIDDEN), jnp.float32),
    mesh=_mesh,
    scratch_types=[
        pltpu.VMEM((NB, BATCH), jnp.int32),
        pltpu.VMEM((NB, BATCH), jnp.int32),
        pltpu.VMEM((BATCH, HIDDEN), jnp.float32),
        pltpu.VMEM_SHARED((N_NODES, HIDDEN), jnp.float32),
        pltpu.SemaphoreType.DMA,
    ],
)
def _sc_edges(h2_hbm, src_hbm, dst_hbm, zeros_hbm, out_hbm,
              src_v, dst_v, rows_v, acc_sh, sem):
    cid = lax.axis_index("c")
    sid = lax.axis_index("s")
    wid = sid * NC + cid
    pltpu.sync_copy(src_hbm.at[wid], src_v)
    pltpu.sync_copy(dst_hbm.at[wid], dst_v)
    rows = pl.ds(sid * RPT, RPT)
    pltpu.sync_copy(zeros_hbm.at[rows], acc_sh.at[rows])
    plsc.subcore_barrier()

    def body(i, carry):
        pltpu.async_copy(h2_hbm.at[src_v.at[i]], rows_v, sem).wait()
        pltpu.sync_copy(rows_v, acc_sh.at[dst_v.at[i]], add=True)
        return carry

    lax.fori_loop(0, NB, body, 0)
    plsc.subcore_barrier()
    pltpu.sync_copy(acc_sh.at[rows], out_hbm.at[cid, rows])


def _tc_h2_body(x_ref, w_ref, degp_ref, h2_ref, dis_ref):
    deg = degp_ref[0][:, :1] + degp_ref[1][:, :1] + 1.0
    dis = lax.rsqrt(deg)
    h = jnp.dot(x_ref[...], w_ref[...], preferred_element_type=jnp.float32)
    h2_ref[...] = h * dis
    dis_ref[...] = dis


_tc_h2 = pl.pallas_call(
    _tc_h2_body,
    out_shape=(
        jax.ShapeDtypeStruct((N_NODES, HIDDEN), jnp.float32),
        jax.ShapeDtypeStruct((N_NODES, 1), jnp.float32),
    ),
)


def _tc_out_body(acc_ref, h2_ref, dis_ref, bg_ref, wc_ref, bc_ref, out_ref):
    s = (acc_ref[0] + acc_ref[1] + h2_ref[...]) * dis_ref[...] + bg_ref[...]
    g = jnp.maximum(s, 0.0)
    out_ref[...] = (
        jnp.dot(g, wc_ref[...], preferred_element_type=jnp.float32)
        + bc_ref[...])


_tc_out = pl.pallas_call(
    _tc_out_body,
    out_shape=jax.ShapeDtypeStruct((N_NODES, NUM_CLASSES), jnp.float32),
)


def kernel(x, edge_index, W_gcn, b_gcn, W_cls, b_cls):
    ei = edge_index.astype(jnp.int32)
    src3 = ei[0].reshape(NW, NB, BATCH)
    dst3 = ei[1].reshape(NW, NB, BATCH)
    ones_deg = jnp.ones((BATCH, DEGW), jnp.float32)
    zeros_deg = jnp.zeros((N_NODES, DEGW), jnp.float32)
    zeros_acc = jnp.zeros((N_NODES, HIDDEN), jnp.float32)

    degp = _sc_degree(dst3, ones_deg, zeros_deg)
    h2, dis = _tc_h2(x, W_gcn, degp)
    acc = _sc_edges(h2, src3, dst3, zeros_acc)
    logits = _tc_out(acc, h2, dis,
                     b_gcn.reshape(1, HIDDEN),
                     W_cls, b_cls.reshape(1, NUM_CLASSES))
    return logits


# speedup vs baseline: 31.0463x; 31.0463x over previous
import functools

import jax
import jax.numpy as jnp
from jax import lax
from jax.experimental import pallas as pl
from jax.experimental.pallas import tpu as pltpu
from jax.experimental.pallas import tpu_sc as plsc

N_NODES = 10000
IN_DIM = 128
HIDDEN = 64
NUM_CLASSES = 7
N_EDGES = 320000

NC = 2
NS = 16
NW = NC * NS
EPW = N_EDGES // NW
BATCH = 80
NB = EPW // BATCH
NP = 10240
RPT = NP // NS
DEGW = 16

_mesh = plsc.VectorSubcoreMesh(
    core_axis_name="c", subcore_axis_name="s", num_cores=NC, num_subcores=NS)


def _sc_degree_body(dst_hbm, ones_hbm, zeros_hbm, out_hbm, idx_v, ones_v, acc_sh):
    cid = lax.axis_index("c")
    sid = lax.axis_index("s")
    wid = sid * NC + cid
    pltpu.sync_copy(dst_hbm.at[wid], idx_v)
    pltpu.sync_copy(ones_hbm, ones_v)
    rows = pl.ds(sid * RPT, RPT)
    pltpu.sync_copy(zeros_hbm.at[rows], acc_sh.at[rows])
    plsc.subcore_barrier()

    def body(i, carry):
        pltpu.sync_copy(ones_v, acc_sh.at[idx_v.at[i]], add=True)
        return carry

    lax.fori_loop(0, NB, body, 0)
    plsc.subcore_barrier()
    pltpu.sync_copy(acc_sh.at[rows], out_hbm.at[cid, rows])


_sc_degree = pl.kernel(
    _sc_degree_body,
    out_type=jax.ShapeDtypeStruct((NC, NP, DEGW), jnp.float32),
    mesh=_mesh,
    scratch_types=[
        pltpu.VMEM((NB, BATCH), jnp.int32),
        pltpu.VMEM((BATCH, DEGW), jnp.float32),
        pltpu.VMEM_SHARED((NP, DEGW), jnp.float32),
    ],
    compiler_params=pltpu.CompilerParams(use_tc_tiling_on_sc=False),
)


def _sc_edges_body(h2_hbm, src_hbm, dst_hbm, zeros_hbm, out_hbm,
                   src_v, dst_v, rows_v, acc_sh, sem):
    cid = lax.axis_index("c")
    sid = lax.axis_index("s")
    wid = sid * NC + cid
    pltpu.sync_copy(src_hbm.at[wid], src_v)
    pltpu.sync_copy(dst_hbm.at[wid], dst_v)
    rows = pl.ds(sid * RPT, RPT)
    pltpu.sync_copy(zeros_hbm.at[rows], acc_sh.at[rows])
    plsc.subcore_barrier()

    def body(i, carry):
        pltpu.async_copy(h2_hbm.at[src_v.at[i]], rows_v, sem).wait()
        pltpu.sync_copy(rows_v, acc_sh.at[dst_v.at[i]], add=True)
        return carry

    lax.fori_loop(0, NB, body, 0)
    plsc.subcore_barrier()
    pltpu.sync_copy(acc_sh.at[rows], out_hbm.at[cid, rows])


_sc_edges = pl.kernel(
    _sc_edges_body,
    out_type=jax.ShapeDtypeStruct((NC, NP, HIDDEN), jnp.float32),
    mesh=_mesh,
    scratch_types=[
        pltpu.VMEM((NB, BATCH), jnp.int32),
        pltpu.VMEM((NB, BATCH), jnp.int32),
        pltpu.VMEM((BATCH, HIDDEN), jnp.float32),
        pltpu.VMEM_SHARED((NP, HIDDEN), jnp.float32),
        pltpu.SemaphoreType.DMA,
    ],
    compiler_params=pltpu.CompilerParams(use_tc_tiling_on_sc=False),
)


def _tc_h2_body(x_ref, w_ref, degp_ref, h2_ref, dis_ref):
    deg = degp_ref[0][:, :1] + degp_ref[1][:, :1] + 1.0
    dis = lax.rsqrt(deg)
    h = jnp.dot(x_ref[...], w_ref[...], preferred_element_type=jnp.float32)
    h2_ref[...] = h * dis
    dis_ref[...] = dis


_tc_h2 = pl.pallas_call(
    _tc_h2_body,
    out_shape=(
        jax.ShapeDtypeStruct((NP, HIDDEN), jnp.float32),
        jax.ShapeDtypeStruct((NP, 1), jnp.float32),
    ),
)


def _tc_out_body(acc_ref, h2_ref, dis_ref, bg_ref, wc_ref, bc_ref, out_ref):
    s = (acc_ref[0] + acc_ref[1] + h2_ref[...]) * dis_ref[...] + bg_ref[...]
    g = jnp.maximum(s, 0.0)
    out_ref[...] = (
        jnp.dot(g, wc_ref[...], preferred_element_type=jnp.float32)
        + bc_ref[...])


_tc_out = pl.pallas_call(
    _tc_out_body,
    out_shape=jax.ShapeDtypeStruct((NP, NUM_CLASSES), jnp.float32),
)


def kernel(x, edge_index, W_gcn, b_gcn, W_cls, b_cls):
    ei = edge_index.astype(jnp.int32)
    src3 = ei[0].reshape(NW, NB, BATCH)
    dst3 = ei[1].reshape(NW, NB, BATCH)
    ones_deg = jnp.ones((BATCH, DEGW), jnp.float32)
    zeros_deg = jnp.zeros((NP, DEGW), jnp.float32)
    zeros_acc = jnp.zeros((NP, HIDDEN), jnp.float32)
    xp = jnp.concatenate([x, jnp.zeros((NP - N_NODES, IN_DIM), x.dtype)])

    degp = _sc_degree(dst3, ones_deg, zeros_deg)
    h2, dis = _tc_h2(xp, W_gcn, degp)
    acc = _sc_edges(h2, src3, dst3, zeros_acc)
    logits = _tc_out(acc, h2, dis,
                     b_gcn.reshape(1, HIDDEN),
                     W_cls, b_cls.reshape(1, NUM_CLASSES))
    return logits[:N_NODES]


# speedup vs baseline: 41.9556x; 1.3514x over previous
import functools

import jax
import jax.numpy as jnp
from jax import lax
from jax.experimental import pallas as pl
from jax.experimental.pallas import tpu as pltpu
from jax.experimental.pallas import tpu_sc as plsc

N_NODES = 10000
IN_DIM = 128
HIDDEN = 64
NUM_CLASSES = 7
N_EDGES = 320000

NC = 2
NS = 16
NW = NC * NS
EPW = N_EDGES // NW
BATCH = 80
NB = EPW // BATCH
NP = 10240
RPT = NP // NS
DEGW = 16

_mesh = plsc.VectorSubcoreMesh(
    core_axis_name="c", subcore_axis_name="s", num_cores=NC, num_subcores=NS)


def _sc_degree_body(dst_hbm, ones_hbm, zeros_hbm, out_hbm, idx_v, ones_v, acc_sh):
    cid = lax.axis_index("c")
    sid = lax.axis_index("s")
    wid = sid * NC + cid
    pltpu.sync_copy(dst_hbm.at[wid], idx_v)
    pltpu.sync_copy(ones_hbm, ones_v)
    rows = pl.ds(sid * RPT, RPT)
    pltpu.sync_copy(zeros_hbm.at[rows], acc_sh.at[rows])
    plsc.subcore_barrier()

    def body(i, carry):
        pltpu.sync_copy(ones_v, acc_sh.at[idx_v.at[i]], add=True)
        return carry

    lax.fori_loop(0, NB, body, 0)
    plsc.subcore_barrier()
    pltpu.sync_copy(acc_sh.at[rows], out_hbm.at[cid, rows])


_sc_degree = pl.kernel(
    _sc_degree_body,
    out_type=jax.ShapeDtypeStruct((NC, NP, DEGW), jnp.float32),
    mesh=_mesh,
    scratch_types=[
        pltpu.VMEM((NB, BATCH), jnp.int32),
        pltpu.VMEM((BATCH, DEGW), jnp.float32),
        pltpu.VMEM_SHARED((NP, DEGW), jnp.float32),
    ],
    compiler_params=pltpu.CompilerParams(use_tc_tiling_on_sc=False),
)


def _sc_edges_body(h2_hbm, src_hbm, dst_hbm, zeros_hbm, out_hbm,
                   src_v, dst_v, rows_a, rows_b, acc_sh, sem_a, sem_b):
    cid = lax.axis_index("c")
    sid = lax.axis_index("s")
    wid = sid * NC + cid
    pltpu.sync_copy(src_hbm.at[wid], src_v)
    pltpu.sync_copy(dst_hbm.at[wid], dst_v)
    rows = pl.ds(sid * RPT, RPT)
    pltpu.sync_copy(zeros_hbm.at[rows], acc_sh.at[rows])
    plsc.subcore_barrier()

    def gather(i, buf, sem):
        pltpu.async_copy(h2_hbm.at[src_v.at[i]], buf, sem)

    def gwait(buf, sem):
        pltpu.make_async_copy(h2_hbm.at[src_v.at[0]], buf, sem).wait()

    def scatter(i, buf):
        pltpu.sync_copy(buf, acc_sh.at[dst_v.at[i]], add=True)

    gather(0, rows_a, sem_a)

    def body(io, carry):
        b0 = 2 * io
        gather(b0 + 1, rows_b, sem_b)
        gwait(rows_a, sem_a)
        scatter(b0, rows_a)
        gather(b0 + 2, rows_a, sem_a)
        gwait(rows_b, sem_b)
        scatter(b0 + 1, rows_b)
        return carry

    lax.fori_loop(0, (NB - 1) // 2, body, 0)
    gwait(rows_a, sem_a)
    scatter(NB - 1, rows_a)
    plsc.subcore_barrier()
    pltpu.sync_copy(acc_sh.at[rows], out_hbm.at[cid, rows])


_sc_edges = pl.kernel(
    _sc_edges_body,
    out_type=jax.ShapeDtypeStruct((NC, NP, HIDDEN), jnp.float32),
    mesh=_mesh,
    scratch_types=[
        pltpu.VMEM((NB, BATCH), jnp.int32),
        pltpu.VMEM((NB, BATCH), jnp.int32),
        pltpu.VMEM((BATCH, HIDDEN), jnp.float32),
        pltpu.VMEM((BATCH, HIDDEN), jnp.float32),
        pltpu.VMEM_SHARED((NP, HIDDEN), jnp.float32),
        pltpu.SemaphoreType.DMA,
        pltpu.SemaphoreType.DMA,
    ],
    compiler_params=pltpu.CompilerParams(use_tc_tiling_on_sc=False),
)


def _tc_h2_body(x_ref, w_ref, degp_ref, h2_ref, dis_ref):
    deg = degp_ref[0][:, :1] + degp_ref[1][:, :1] + 1.0
    dis = lax.rsqrt(deg)
    h = jnp.dot(x_ref[...], w_ref[...], preferred_element_type=jnp.float32)
    h2_ref[:N_NODES] = h * dis[:N_NODES]
    h2_ref[N_NODES:] = jnp.zeros((NP - N_NODES, HIDDEN), jnp.float32)
    dis_ref[...] = dis


_tc_h2 = pl.pallas_call(
    _tc_h2_body,
    out_shape=(
        jax.ShapeDtypeStruct((NP, HIDDEN), jnp.float32),
        jax.ShapeDtypeStruct((NP, 1), jnp.float32),
    ),
)


def _tc_out_body(acc_ref, h2_ref, dis_ref, bg_ref, wc_ref, bc_ref, out_ref):
    s = (acc_ref[0] + acc_ref[1] + h2_ref[...]) * dis_ref[...] + bg_ref[...]
    g = jnp.maximum(s, 0.0)
    out_ref[...] = (
        jnp.dot(g, wc_ref[...], preferred_element_type=jnp.float32)
        + bc_ref[...])


_tc_out = pl.pallas_call(
    _tc_out_body,
    out_shape=jax.ShapeDtypeStruct((NP, NUM_CLASSES), jnp.float32),
)


def kernel(x, edge_index, W_gcn, b_gcn, W_cls, b_cls):
    ei = edge_index.astype(jnp.int32)
    src3 = ei[0].reshape(NW, NB, BATCH)
    dst3 = ei[1].reshape(NW, NB, BATCH)
    ones_deg = jnp.ones((BATCH, DEGW), jnp.float32)
    zeros_deg = jnp.zeros((NP, DEGW), jnp.float32)
    zeros_acc = jnp.zeros((NP, HIDDEN), jnp.float32)

    degp = _sc_degree(dst3, ones_deg, zeros_deg)
    h2, dis = _tc_h2(x, W_gcn, degp)
    acc = _sc_edges(h2, src3, dst3, zeros_acc)
    logits = _tc_out(acc, h2, dis,
                     b_gcn.reshape(1, HIDDEN),
                     W_cls, b_cls.reshape(1, NUM_CLASSES))
    return logits[:N_NODES]
